# baseline (speedup 1.0000x reference)
import functools

import jax
import jax.numpy as jnp
from jax.experimental import pallas as pl
from jax.experimental.pallas import tpu as pltpu

_TOPK = 64
_SEARCH_ITERS = 16


def _enc_body(nw, wc_cols, topk,
              x_ref, w_ref, be_ref, z_ref, acc_ref, thr_ref):
    wc = pl.program_id(1)

    @pl.when(wc < nw)
    def _matmul():
        prod = jax.lax.dot_general(
            x_ref[...], w_ref[...], (((1,), (1,)), ((), ())),
            preferred_element_type=jnp.float32)
        acc_ref[wc] = prod + be_ref[0, pl.ds(wc * wc_cols, wc_cols)][None, :]

    @pl.when(wc == nw - 1)
    def _search():
        rb_rows = acc_ref.shape[1]
        zero = jnp.zeros((rb_rows, 1), jnp.float32)

        def rowmax(j, m):
            return jnp.maximum(m, jnp.max(acc_ref[j], axis=1, keepdims=True))

        hi = jax.lax.fori_loop(0, nw, rowmax, zero)
        lo = zero

        def it(_, lh):
            lo, hi = lh
            w = (hi - lo) * (1.0 / 3.0)
            m1 = lo + w
            m2 = hi - w

            def cchunk(j, c):
                a = acc_ref[j]
                c1 = c[0] + jnp.sum((a >= m1).astype(jnp.float32),
                                    axis=1, keepdims=True)
                c2 = c[1] + jnp.sum((a >= m2).astype(jnp.float32),
                                    axis=1, keepdims=True)
                return (c1, c2)

            c1, c2 = jax.lax.fori_loop(0, nw, cchunk, (zero, zero))
            g1 = c1 >= float(topk)
            g2 = c2 >= float(topk)
            new_lo = jnp.where(g2, m2, jnp.where(g1, m1, lo))
            new_hi = jnp.where(g2, hi, jnp.where(g1, m2, m1))
            return new_lo, new_hi

        lo, hi = jax.lax.fori_loop(0, _SEARCH_ITERS, it, (lo, hi))
        thr_ref[...] = lo

    @pl.when(wc >= nw)
    def _write_z():
        a = acc_ref[wc - nw]
        z_ref[...] = jnp.where(a >= thr_ref[...],
                               jnp.maximum(a, 0.0), 0.0)


def _dec_body(nk, z_ref, w_ref, bd_ref, xh_ref, acc_ref):
    kc = pl.program_id(0)

    @pl.when(kc == 0)
    def _init():
        acc_ref[...] = jnp.zeros_like(acc_ref)

    zb = z_ref[...].astype(jnp.bfloat16)
    acc_ref[...] += jax.lax.dot_general(
        zb, w_ref[...], (((1,), (0,)), ((), ())),
        preferred_element_type=jnp.float32)

    @pl.when(kc == nk - 1)
    def _fin():
        xh_ref[...] = acc_ref[...] + bd_ref[...]


def _loss_body(denom, x_ref, xh_ref, out_ref):
    d = xh_ref[...] - x_ref[...]
    out_ref[...] = (jnp.sum(d * d) * (1.0 / denom)).reshape(1, 1)


def kernel(x, W_dec, b_enc, b_dec):
    B, T, D_IN = x.shape
    D_SAE = W_dec.shape[0]
    d_flat = T * D_IN

    xf = x.reshape(B, d_flat)
    xb = xf.astype(jnp.bfloat16)
    Wb = W_dec.reshape(D_SAE, d_flat).astype(jnp.bfloat16)
    be2 = b_enc.reshape(1, D_SAE)
    bd2 = b_dec.reshape(1, d_flat)

    RB = min(512, B)
    WC = min(512, D_SAE)
    NW = D_SAE // WC
    NZ = NW

    z = pl.pallas_call(
        functools.partial(_enc_body, NW, WC, _TOPK),
        grid=(B // RB, NW + NZ),
        in_specs=[
            pl.BlockSpec((RB, d_flat), lambda rb, wc: (rb, 0)),
            pl.BlockSpec((WC, d_flat),
                         lambda rb, wc: (jnp.minimum(wc, NW - 1), 0)),
            pl.BlockSpec((1, D_SAE), lambda rb, wc: (0, 0)),
        ],
        out_specs=pl.BlockSpec(
            (RB, WC), lambda rb, wc: (rb, jnp.maximum(wc - NW, 0))),
        out_shape=jax.ShapeDtypeStruct((B, D_SAE), jnp.float32),
        scratch_shapes=[
            pltpu.VMEM((NW, RB, WC), jnp.float32),
            pltpu.VMEM((RB, 1), jnp.float32),
        ],
    )(xb, Wb, be2)

    KC = min(1024, D_SAE)
    NK = D_SAE // KC
    xh = pl.pallas_call(
        functools.partial(_dec_body, NK),
        grid=(NK,),
        in_specs=[
            pl.BlockSpec((B, KC), lambda kc: (0, kc)),
            pl.BlockSpec((KC, d_flat), lambda kc: (kc, 0)),
            pl.BlockSpec((1, d_flat), lambda kc: (0, 0)),
        ],
        out_specs=pl.BlockSpec((B, d_flat), lambda kc: (0, 0)),
        out_shape=jax.ShapeDtypeStruct((B, d_flat), jnp.float32),
        scratch_shapes=[pltpu.VMEM((B, d_flat), jnp.float32)],
    )(z, Wb, bd2)

    lossm = pl.pallas_call(
        functools.partial(_loss_body, float(B * T)),
        grid=(1,),
        in_specs=[
            pl.BlockSpec((B, d_flat), lambda i: (0, 0)),
            pl.BlockSpec((B, d_flat), lambda i: (0, 0)),
        ],
        out_specs=pl.BlockSpec((1, 1), lambda i: (0, 0)),
        out_shape=jax.ShapeDtypeStruct((1, 1), jnp.float32),
    )(xf, xh)

    return (lossm[0, 0], xh.reshape(B, T, D_IN), z)


# speedup vs baseline: 1.0765x; 1.0765x over previous
import functools

import jax
import jax.numpy as jnp
from jax import lax
from jax.experimental import pallas as pl
from jax.experimental.pallas import tpu as pltpu
from jax.experimental.pallas import tpu_sc as plsc

_TOPK = 64
_SEARCH_ITERS = 22


def _enc_body(nw, wc_cols, topk,
              x_ref, w_ref, be_ref, pre_ref, thr_ref, acc_ref):
    wc = pl.program_id(1)

    prod = jax.lax.dot_general(
        x_ref[...], w_ref[...], (((1,), (1,)), ((), ())),
        preferred_element_type=jnp.float32)
    prod = prod + be_ref[0, pl.ds(wc * wc_cols, wc_cols)][None, :]
    acc_ref[wc] = prod
    pre_ref[...] = prod

    @pl.when(wc == nw - 1)
    def _search():
        rb_rows = acc_ref.shape[1]
        zero = jnp.zeros((rb_rows, 1), jnp.float32)

        def rowmax(j, m):
            return jnp.maximum(m, jnp.max(acc_ref[j], axis=1, keepdims=True))

        hi = jax.lax.fori_loop(0, nw, rowmax, zero)
        lo = zero

        def it(_, lh):
            lo, hi = lh
            mid = 0.5 * (lo + hi)

            def cchunk(j, c):
                return c + jnp.sum(
                    (acc_ref[j] >= mid).astype(jnp.float32),
                    axis=1, keepdims=True)

            cnt = jax.lax.fori_loop(0, nw, cchunk, zero)
            ge = cnt >= float(topk)
            return jnp.where(ge, mid, lo), jnp.where(ge, hi, mid)

        lo, hi = jax.lax.fori_loop(0, _SEARCH_ITERS, it, (lo, hi))
        thr_ref[...] = lo


def _dec_body(nk, pre_ref, thr_ref, w_ref, bd_ref, xh_ref, acc_ref):
    kc = pl.program_id(0)

    @pl.when(kc == 0)
    def _init():
        acc_ref[...] = jnp.zeros_like(acc_ref)

    p = pre_ref[...]
    zb = jnp.where(p >= thr_ref[...],
                   jnp.maximum(p, 0.0), 0.0).astype(jnp.bfloat16)
    acc_ref[...] += jax.lax.dot_general(
        zb, w_ref[...], (((1,), (0,)), ((), ())),
        preferred_element_type=jnp.float32)

    @pl.when(kc == nk - 1)
    def _fin():
        xh_ref[...] = acc_ref[...] + bd_ref[...]


def _z_sc_body(rows_per_w, d_sae,
               pre_hbm, thr16_hbm, z_hbm, buf_in, buf_out, thr_v):
    wid = lax.axis_index("s") * 2 + lax.axis_index("c")
    base = wid * rows_per_w

    def row_body(r, carry):
        row = base + r
        pltpu.sync_copy(pre_hbm.at[row], buf_in)
        pltpu.sync_copy(thr16_hbm.at[row], thr_v)
        tvec = thr_v[...]

        def vec_body(j, c):
            v = buf_in[pl.ds(j * 16, 16)]
            buf_out[pl.ds(j * 16, 16)] = jnp.where(
                v >= tvec, jnp.maximum(v, 0.0), 0.0)
            return c

        jax.lax.fori_loop(0, d_sae // 16, vec_body, 0)
        pltpu.sync_copy(buf_out, z_hbm.at[row])
        return carry

    jax.lax.fori_loop(0, rows_per_w, row_body, 0)


def _loss_body(denom, x_ref, xh_ref, out_ref):
    d = xh_ref[...] - x_ref[...]
    out_ref[...] = (jnp.sum(d * d) * (1.0 / denom)).reshape(1, 1)


def kernel(x, W_dec, b_enc, b_dec):
    B, T, D_IN = x.shape
    D_SAE = W_dec.shape[0]
    d_flat = T * D_IN

    xf = x.reshape(B, d_flat)
    xb = xf.astype(jnp.bfloat16)
    Wb = W_dec.reshape(D_SAE, d_flat).astype(jnp.bfloat16)
    be2 = b_enc.reshape(1, D_SAE)
    bd2 = b_dec.reshape(1, d_flat)

    RB = min(512, B)
    WC = min(512, D_SAE)
    NW = D_SAE // WC

    pre, thr = pl.pallas_call(
        functools.partial(_enc_body, NW, WC, _TOPK),
        grid=(B // RB, NW),
        in_specs=[
            pl.BlockSpec((RB, d_flat), lambda rb, wc: (rb, 0)),
            pl.BlockSpec((WC, d_flat), lambda rb, wc: (wc, 0)),
            pl.BlockSpec((1, D_SAE), lambda rb, wc: (0, 0)),
        ],
        out_specs=[
            pl.BlockSpec((RB, WC), lambda rb, wc: (rb, wc)),
            pl.BlockSpec((RB, 1), lambda rb, wc: (rb, 0)),
        ],
        out_shape=[
            jax.ShapeDtypeStruct((B, D_SAE), jnp.float32),
            jax.ShapeDtypeStruct((B, 1), jnp.float32),
        ],
        scratch_shapes=[
            pltpu.VMEM((NW, RB, WC), jnp.float32),
        ],
    )(xb, Wb, be2)

    KC = min(512, D_SAE)
    NK = D_SAE // KC
    xh = pl.pallas_call(
        functools.partial(_dec_body, NK),
        grid=(NK,),
        in_specs=[
            pl.BlockSpec((B, KC), lambda kc: (0, kc)),
            pl.BlockSpec((B, 1), lambda kc: (0, 0)),
            pl.BlockSpec((KC, d_flat), lambda kc: (kc, 0)),
            pl.BlockSpec((1, d_flat), lambda kc: (0, 0)),
        ],
        out_specs=pl.BlockSpec((B, d_flat), lambda kc: (0, 0)),
        out_shape=jax.ShapeDtypeStruct((B, d_flat), jnp.float32),
        scratch_shapes=[pltpu.VMEM((B, d_flat), jnp.float32)],
    )(pre, thr, Wb, bd2)

    n_workers = 32
    rows_per_w = B // n_workers
    mesh = plsc.VectorSubcoreMesh(core_axis_name="c", subcore_axis_name="s")
    z = pl.kernel(
        functools.partial(_z_sc_body, rows_per_w, D_SAE),
        mesh=mesh,
        out_type=jax.ShapeDtypeStruct((B, D_SAE), jnp.float32),
        scratch_types=[
            pltpu.VMEM((D_SAE,), jnp.float32),
            pltpu.VMEM((D_SAE,), jnp.float32),
            pltpu.VMEM((16,), jnp.float32),
        ],
    )(pre, jnp.broadcast_to(thr, (B, 16)))

    lossm = pl.pallas_call(
        functools.partial(_loss_body, float(B * T)),
        grid=(1,),
        in_specs=[
            pl.BlockSpec((B, d_flat), lambda i: (0, 0)),
            pl.BlockSpec((B, d_flat), lambda i: (0, 0)),
        ],
        out_specs=pl.BlockSpec((1, 1), lambda i: (0, 0)),
        out_shape=jax.ShapeDtypeStruct((1, 1), jnp.float32),
    )(xf, xh)

    return (lossm[0, 0], xh.reshape(B, T, D_IN), z)
